# baseline (speedup 1.0000x reference)
import numpy as np
import jax
import jax.numpy as jnp
from jax import lax
from jax.experimental import pallas as pl
from jax.experimental.pallas import tpu as pltpu
from jax.experimental.pallas import tpu_sc as plsc

_R = 512
_NW = 32
_N = 262144
_RPW = _N // _NW

_INV_PIO2 = float(2.0 / np.pi)
_PIO2_HI = float(np.float32(np.round(np.pi / 2 * 2.0**17) / 2.0**17))
_rem = np.pi / 2 - _PIO2_HI
_PIO2_MID = float(np.float32(_rem))
_PIO2_LO = float(np.float32(_rem - float(np.float32(_rem))))


def _sincos16(t):
    u = t * _INV_PIO2
    sgn = lax.bitcast_convert_type(u, jnp.int32) & jnp.int32(-2147483648)
    half = lax.bitcast_convert_type(sgn | jnp.int32(1056964608), jnp.float32)
    mi = (u + half).astype(jnp.int32)
    m = mi.astype(jnp.float32)
    r = t - m * _PIO2_HI
    r = r - m * _PIO2_MID
    r = r - m * _PIO2_LO
    r2 = r * r
    sp = r * (1.0 + r2 * (-0.16666654611587524 + r2 * (0.008332160767167807 + r2 * (-0.00019515295891841408))))
    cp = 1.0 + r2 * (-0.5 + r2 * (0.04166664555668831 + r2 * (-0.0013887310633435845 + r2 * 2.476048860093673e-05)))
    qb0 = (mi & 1) == 1
    s_unsigned = jnp.where(qb0, cp, sp)
    c_unsigned = jnp.where(qb0, sp, cp)
    s_flip = jnp.left_shift(mi & 2, 30)
    c_flip = jnp.left_shift((mi + 1) & 2, 30)
    s = lax.bitcast_convert_type(lax.bitcast_convert_type(s_unsigned, jnp.int32) ^ s_flip, jnp.float32)
    c = lax.bitcast_convert_type(lax.bitcast_convert_type(c_unsigned, jnp.int32) ^ c_flip, jnp.float32)
    return s, c


def _sc_call(embedded, t_dirs, embedcam_table, camidx_arr):
    mesh = plsc.VectorSubcoreMesh(core_axis_name="c", subcore_axis_name="s")

    def body(emb_hbm, dirs_hbm, table_hbm, cidx_hbm, out_hbm,
             emb_v, dirs_v, out_v, idx_v, cam_v, sem):
        wid = lax.axis_index("s") * 2 + lax.axis_index("c")
        wbase = wid * _RPW
        pltpu.sync_copy(cidx_hbm, idx_v)
        pltpu.async_copy(table_hbm.at[idx_v], cam_v, sem).wait()
        cam = cam_v[0, :]
        lane = lax.iota(jnp.int32, 16)

        def chunk(ci, carry):
            base = wbase + ci * _R
            rsl = pl.ds(base, _R)
            pltpu.sync_copy(dirs_hbm.at[rsl], dirs_v)
            pltpu.sync_copy(emb_hbm.at[rsl], emb_v)

            def group(g, carry2):
                rows = lane + g * 16
                for c in range(3):
                    xc = plsc.load_gather(dirs_v, [rows, jnp.broadcast_to(jnp.int32(c), (16,))])
                    plsc.store_scatter(out_v, [rows, jnp.broadcast_to(jnp.int32(63 + c), (16,))], xc)
                    s, co = _sincos16(xc)
                    for k in range(4):
                        if k > 0:
                            s, co = 2.0 * s * co, 1.0 - 2.0 * s * s
                        plsc.store_scatter(out_v, [rows, jnp.broadcast_to(jnp.int32(66 + 6 * k + c), (16,))], s)
                        plsc.store_scatter(out_v, [rows, jnp.broadcast_to(jnp.int32(69 + 6 * k + c), (16,))], co)
                return carry2

            lax.fori_loop(0, _R // 16, group, 0)

            def row(r, carry3):
                for o in (0, 16, 32, 47):
                    out_v[r, pl.ds(o, 16)] = emb_v[r, pl.ds(o, 16)]
                out_v[r, pl.ds(90, 16)] = cam
                return carry3

            lax.fori_loop(0, _R, row, 0)
            pltpu.sync_copy(out_v, out_hbm.at[rsl])
            return carry

        lax.fori_loop(0, _RPW // _R, chunk, 0)

    f = pl.kernel(
        body,
        out_type=jax.ShapeDtypeStruct((_N, 106), jnp.float32),
        mesh=mesh,
        compiler_params=pltpu.CompilerParams(
            use_tc_tiling_on_sc=False, needs_layout_passes=False),
        scratch_types=[
            pltpu.VMEM((_R, 63), jnp.float32),
            pltpu.VMEM((_R, 3), jnp.float32),
            pltpu.VMEM((_R, 106), jnp.float32),
            pltpu.VMEM((8,), jnp.int32),
            pltpu.VMEM((8, 16), jnp.float32),
            pltpu.SemaphoreType.DMA,
        ],
    )
    return f(embedded, t_dirs, embedcam_table, camidx_arr)


def kernel(embedded, t_dirs, embedcam_table, camera_idx):
    camidx_arr = jnp.full((8,), camera_idx, dtype=jnp.int32)
    return _sc_call(embedded, t_dirs, embedcam_table, camidx_arr)


# speedup vs baseline: 1.5228x; 1.5228x over previous
import numpy as np
import jax
import jax.numpy as jnp
from jax import lax
from jax.experimental import pallas as pl
from jax.experimental.pallas import tpu as pltpu
from jax.experimental.pallas import tpu_sc as plsc

_R = 256
_NW = 32
_N = 262144
_RPW = _N // _NW

_INV_PIO2 = float(2.0 / np.pi)
_PIO2_HI = float(np.float32(np.round(np.pi / 2 * 2.0**17) / 2.0**17))
_rem = np.pi / 2 - _PIO2_HI
_PIO2_MID = float(np.float32(_rem))
_PIO2_LO = float(np.float32(_rem - float(np.float32(_rem))))


def _sincos16(t):
    u = t * _INV_PIO2
    sgn = lax.bitcast_convert_type(u, jnp.int32) & jnp.int32(-2147483648)
    half = lax.bitcast_convert_type(sgn | jnp.int32(1056964608), jnp.float32)
    mi = (u + half).astype(jnp.int32)
    m = mi.astype(jnp.float32)
    r = t - m * _PIO2_HI
    r = r - m * _PIO2_MID
    r = r - m * _PIO2_LO
    r2 = r * r
    sp = r * (1.0 + r2 * (-0.16666654611587524 + r2 * (0.008332160767167807 + r2 * (-0.00019515295891841408))))
    cp = 1.0 + r2 * (-0.5 + r2 * (0.04166664555668831 + r2 * (-0.0013887310633435845 + r2 * 2.476048860093673e-05)))
    qb0 = (mi & 1) == 1
    s_unsigned = jnp.where(qb0, cp, sp)
    c_unsigned = jnp.where(qb0, sp, cp)
    s_flip = jnp.left_shift(mi & 2, 30)
    c_flip = jnp.left_shift((mi + 1) & 2, 30)
    s = lax.bitcast_convert_type(lax.bitcast_convert_type(s_unsigned, jnp.int32) ^ s_flip, jnp.float32)
    c = lax.bitcast_convert_type(lax.bitcast_convert_type(c_unsigned, jnp.int32) ^ c_flip, jnp.float32)
    return s, c


def _sc_call(embedded, t_dirs, embedcam_table, camidx_arr):
    table_pad = jnp.pad(embedcam_table, ((0, 0), (0, 112)))
    mesh = plsc.VectorSubcoreMesh(core_axis_name="c", subcore_axis_name="s")

    def body(emb_hbm, dirs_hbm, table_hbm, cidx_hbm, out_hbm,
             emb_v, dirs_v, out_v, idx_v, cam_v, sem):
        wid = lax.axis_index("s") * 2 + lax.axis_index("c")
        wbase = wid * _RPW
        lane = lax.iota(jnp.int32, 16)
        pltpu.sync_copy(cidx_hbm, idx_v)
        pltpu.async_copy(table_hbm.at[idx_v], cam_v, sem).wait()
        cam = cam_v[0, pl.ds(0, 16)]

        def chunk(ci, carry):
            base = wbase + ci * _R
            rsl = pl.ds(base, _R)
            pltpu.sync_copy(dirs_hbm.at[rsl], dirs_v)
            pltpu.sync_copy(emb_hbm.at[rsl], emb_v)

            def group(g, carry2):
                rows = lane + g * 16
                for c in range(3):
                    xc = plsc.load_gather(dirs_v, [rows, jnp.broadcast_to(jnp.int32(c), (16,))])
                    plsc.store_scatter(out_v, [rows, jnp.broadcast_to(jnp.int32(63 + c), (16,))], xc)
                    s, co = _sincos16(xc)
                    for k in range(4):
                        if k > 0:
                            s, co = 2.0 * s * co, 1.0 - 2.0 * s * s
                        plsc.store_scatter(out_v, [rows, jnp.broadcast_to(jnp.int32(66 + 6 * k + c), (16,))], s)
                        plsc.store_scatter(out_v, [rows, jnp.broadcast_to(jnp.int32(69 + 6 * k + c), (16,))], co)
                return carry2

            lax.fori_loop(0, _R // 16, group, 0)

            def row(r, carry3):
                for o in (0, 16, 32, 47):
                    out_v[r, pl.ds(o, 16)] = emb_v[r, pl.ds(o, 16)]
                out_v[r, pl.ds(90, 16)] = cam
                return carry3

            lax.fori_loop(0, _R, row, 0)
            pltpu.sync_copy(out_v, out_hbm.at[rsl])
            return carry

        lax.fori_loop(0, _RPW // _R, chunk, 0)

    f = pl.kernel(
        body,
        out_type=jax.ShapeDtypeStruct((_N, 106), jnp.float32),
        mesh=mesh,
        compiler_params=pltpu.CompilerParams(
            use_tc_tiling_on_sc=True, needs_layout_passes=False),
        scratch_types=[
            pltpu.VMEM((_R, 63), jnp.float32),
            pltpu.VMEM((_R, 3), jnp.float32),
            pltpu.VMEM((_R, 106), jnp.float32),
            pltpu.VMEM((8,), jnp.int32),
            pltpu.VMEM((8, 128), jnp.float32),
            pltpu.SemaphoreType.DMA,
        ],
    )
    return f(embedded, t_dirs, table_pad, camidx_arr)


def kernel(embedded, t_dirs, embedcam_table, camera_idx):
    camidx_arr = jnp.full((8,), camera_idx, dtype=jnp.int32)
    return _sc_call(embedded, t_dirs, embedcam_table, camidx_arr)


# speedup vs baseline: 3.1697x; 2.0816x over previous
import numpy as np
import jax
import jax.numpy as jnp
from jax.experimental import pallas as pl
from jax.experimental.pallas import tpu as pltpu

_B = 16384

_INV_PIO2 = float(2.0 / np.pi)
_PIO2_HI = float(np.float32(np.round(np.pi / 2 * 2.0**17) / 2.0**17))
_rem = np.pi / 2 - _PIO2_HI
_PIO2_MID = float(np.float32(_rem))
_PIO2_LO = float(np.float32(_rem - float(np.float32(_rem))))
_RND_MAGIC = float(1.5 * 2.0**23)


def _sincos_rows(t, qoff):
    m = jnp.rint(t * _INV_PIO2)
    mi = m.astype(jnp.int32)
    r = t - m * _PIO2_HI
    r = r - m * _PIO2_MID
    r = r - m * _PIO2_LO
    q = (mi + qoff) & 3
    r2 = r * r
    sp = r * (1.0 + r2 * (-0.16666654611587524 + r2 * (0.008332160767167807 + r2 * (-0.00019515295891841408))))
    cp = 1.0 + r2 * (-0.5 + r2 * (0.04166664555668831 + r2 * (-0.0013887310633435845 + r2 * 2.476048860093673e-05)))
    pick = jnp.where((q & 1) == 1, cp, sp)
    signbit = jnp.left_shift(q & 2, 30)
    return jax.lax.bitcast_convert_type(
        jax.lax.bitcast_convert_type(pick, jnp.int32) ^ signbit, jnp.float32)


def _body(cam_idx_ref, emb_ref, dirst_ref, table_ref, out_ref):
    xT = dirst_ref[...]
    x24T = jnp.concatenate([xT] * 8, axis=0)
    k = jax.lax.broadcasted_iota(jnp.int32, (24, 1), 0) // 3
    f = jnp.left_shift(1, k >> 1).astype(jnp.float32)
    qoff = k & 1
    s24T = _sincos_rows(x24T * f, qoff)
    d27 = jax.lax.transpose(jnp.concatenate([xT, s24T], axis=0), (1, 0))
    cam = table_ref[pl.ds(cam_idx_ref[0], 1), :]
    camb = jnp.broadcast_to(cam, (_B, 16))
    out_ref[...] = jnp.concatenate([emb_ref[...], d27, camb], axis=-1)


def kernel(embedded, t_dirs, embedcam_table, camera_idx):
    N = embedded.shape[0]
    T = embedcam_table.shape[0]
    cam = jnp.asarray(camera_idx, dtype=jnp.int32).reshape((1,))
    dirsT = t_dirs.T
    grid_spec = pltpu.PrefetchScalarGridSpec(
        num_scalar_prefetch=1,
        grid=(N // _B,),
        in_specs=[
            pl.BlockSpec((_B, 63), lambda i, c: (i, 0)),
            pl.BlockSpec((3, _B), lambda i, c: (0, i)),
            pl.BlockSpec((T, 16), lambda i, c: (0, 0)),
        ],
        out_specs=pl.BlockSpec((_B, 106), lambda i, c: (i, 0)),
    )
    return pl.pallas_call(
        _body,
        grid_spec=grid_spec,
        out_shape=jax.ShapeDtypeStruct((N, 106), jnp.float32),
    )(cam, embedded, dirsT, embedcam_table)


# speedup vs baseline: 3.1763x; 1.0021x over previous
import numpy as np
import jax
import jax.numpy as jnp
from jax.experimental import pallas as pl
from jax.experimental.pallas import tpu as pltpu

_B = 16384

_INV_PIO2 = float(2.0 / np.pi)
_PIO2_HI = float(np.float32(np.round(np.pi / 2 * 2.0**17) / 2.0**17))
_rem = np.pi / 2 - _PIO2_HI
_PIO2_MID = float(np.float32(_rem))
_PIO2_LO = float(np.float32(_rem - float(np.float32(_rem))))


def _sincos_rows(t, qoff):
    m = jnp.rint(t * _INV_PIO2)
    mi = m.astype(jnp.int32)
    r = t - m * _PIO2_HI
    r = r - m * _PIO2_MID
    r = r - m * _PIO2_LO
    q = (mi + qoff) & 3
    r2 = r * r
    sp = r * (1.0 + r2 * (-0.16666654611587524 + r2 * (0.008332160767167807 + r2 * (-0.00019515295891841408))))
    cp = 1.0 + r2 * (-0.5 + r2 * (0.04166664555668831 + r2 * (-0.0013887310633435845 + r2 * 2.476048860093673e-05)))
    pick = jnp.where((q & 1) == 1, cp, sp)
    signbit = jnp.left_shift(q & 2, 30)
    return jax.lax.bitcast_convert_type(
        jax.lax.bitcast_convert_type(pick, jnp.int32) ^ signbit, jnp.float32)


def _body(cam_idx_ref, emb_ref, dirst_ref, table_ref, out_ref):
    xT = dirst_ref[...]
    x24T = jnp.concatenate([xT] * 8, axis=0)
    k = jax.lax.broadcasted_iota(jnp.int32, (24, 1), 0) // 3
    f = jnp.left_shift(1, k >> 1).astype(jnp.float32)
    qoff = k & 1
    s24T = _sincos_rows(x24T * f, qoff)
    d27 = jax.lax.transpose(jnp.concatenate([xT, s24T], axis=0), (1, 0))
    cam = table_ref[pl.ds(cam_idx_ref[0], 1), :]
    camb = jnp.broadcast_to(cam, (_B, 16))
    out_ref[...] = jnp.concatenate([emb_ref[...], d27, camb], axis=-1)


def kernel(embedded, t_dirs, embedcam_table, camera_idx):
    N = embedded.shape[0]
    T = embedcam_table.shape[0]
    cam = jnp.asarray(camera_idx, dtype=jnp.int32).reshape((1,))
    dirsT = t_dirs.T
    grid_spec = pltpu.PrefetchScalarGridSpec(
        num_scalar_prefetch=1,
        grid=(N // _B,),
        in_specs=[
            pl.BlockSpec((_B, 63), lambda i, c: (i, 0)),
            pl.BlockSpec((3, _B), lambda i, c: (0, i)),
            pl.BlockSpec((T, 16), lambda i, c: (0, 0)),
        ],
        out_specs=pl.BlockSpec((_B, 106), lambda i, c: (i, 0)),
    )
    return pl.pallas_call(
        _body,
        grid_spec=grid_spec,
        out_shape=jax.ShapeDtypeStruct((N, 106), jnp.float32),
    )(cam, embedded, dirsT, embedcam_table)
